# initial kernel scaffold (unmeasured)
import jax
import jax.numpy as jnp
from jax import lax
from jax.experimental import pallas as pl
from jax.experimental.pallas import tpu as pltpu

N_DEV = 8
M, N = 4096, 8192
MC = M // N_DEV


def _ar_body(partial_ref, sx_ref, sw_ref, out_ref,
             buf_ref, pchunk_ref, send_sems, recv_sems,
             local_sems, credit_sem):
    my = lax.axis_index("i")
    left = lax.rem(my + N_DEV - 1, N_DEV)
    right = lax.rem(my + 1, N_DEV)

    barrier = pltpu.get_barrier_semaphore()
    for nbr in (left, right):
        pl.semaphore_signal(barrier, inc=1, device_id=(nbr,),
                            device_id_type=pl.DeviceIdType.MESH)
    pl.semaphore_wait(barrier, 2)

    seed = pltpu.make_async_copy(
        partial_ref.at[pl.ds(my * MC, MC), :], buf_ref.at[0],
        local_sems.at[0])
    seed.start()
    seed.wait()

    def ring_send(g):
        rdma = pltpu.make_async_remote_copy(
            src_ref=buf_ref.at[g % 2],
            dst_ref=buf_ref.at[(g + 1) % 2],
            send_sem=send_sems.at[g % 2],
            recv_sem=recv_sems.at[(g + 1) % 2],
            device_id=(right,),
            device_id_type=pl.DeviceIdType.MESH,
        )
        if g >= 1:
            pl.semaphore_wait(credit_sem, 1)
        rdma.start()
        return rdma

    def ring_credit(g):
        if g < 2 * (N_DEV - 1) - 1:
            pl.semaphore_signal(credit_sem, inc=1, device_id=(left,),
                                device_id_type=pl.DeviceIdType.MESH)

    for s in range(N_DEV - 1):
        rdma = ring_send(s)
        c = lax.rem(my + (N_DEV - 1 - s), N_DEV)
        load = pltpu.make_async_copy(
            partial_ref.at[pl.ds(c * MC, MC), :], pchunk_ref,
            local_sems.at[0])
        load.start()
        rdma.wait_recv()
        load.wait()
        buf_ref[(s + 1) % 2] = buf_ref[(s + 1) % 2] + pchunk_ref[...]
        rdma.wait_send()
        ring_credit(s)

    own = lax.rem(my + 1, N_DEV)
    scale = sx_ref[0] * sw_ref[0]
    buf_ref[1] = jnp.maximum(buf_ref[1] * scale, 0.0)
    st = pltpu.make_async_copy(
        buf_ref.at[1], out_ref.at[pl.ds(own * MC, MC), :], local_sems.at[1])
    st.start()
    st.wait()

    for t in range(N_DEV - 1):
        g = (N_DEV - 1) + t
        rdma = ring_send(g)
        cr = lax.rem(my + (N_DEV - t), N_DEV)
        rdma.wait_recv()
        st = pltpu.make_async_copy(
            buf_ref.at[(g + 1) % 2], out_ref.at[pl.ds(cr * MC, MC), :],
            local_sems.at[1])
        st.start()
        st.wait()
        rdma.wait_send()
        ring_credit(g)


def _allreduce(partial, scale_x, scale_w):
    return pl.pallas_call(
        _ar_body,
        out_shape=jax.ShapeDtypeStruct((M, N), jnp.float32),
        in_specs=[
            pl.BlockSpec(memory_space=pltpu.ANY),
            pl.BlockSpec(memory_space=pltpu.SMEM),
            pl.BlockSpec(memory_space=pltpu.SMEM),
        ],
        out_specs=pl.BlockSpec(memory_space=pltpu.ANY),
        scratch_shapes=[
            pltpu.VMEM((2, MC, N), jnp.float32),
            pltpu.VMEM((MC, N), jnp.float32),
            pltpu.SemaphoreType.DMA((2,)),
            pltpu.SemaphoreType.DMA((2,)),
            pltpu.SemaphoreType.DMA((2,)),
            pltpu.SemaphoreType.REGULAR,
        ],
        compiler_params=pltpu.CompilerParams(collective_id=0),
    )(partial, scale_x, scale_w)


def kernel(x, w_mat, scale_x, scale_w):
    partial = lax.dot_general(
        x.astype(jnp.bfloat16), w_mat.astype(jnp.bfloat16),
        (((1,), (0,)), ((), ())), preferred_element_type=jnp.float32)
    return _allreduce(partial, scale_x.astype(jnp.float32),
                      scale_w.astype(jnp.float32))


# baseline (device time: 2770137 ns/iter reference)
import jax
import jax.numpy as jnp
from jax import lax
from jax.experimental import pallas as pl
from jax.experimental.pallas import tpu as pltpu

N_DEV = 8
M, N = 4096, 8192
MC = M // N_DEV


def _ar_body(partial_ref, sx_ref, sw_ref, out_ref,
             buf_ref, pchunk_ref, send_sems, recv_sems,
             local_sems, credit_sem):
    my = lax.axis_index("i")
    left = lax.rem(my + N_DEV - 1, N_DEV)
    right = lax.rem(my + 1, N_DEV)

    barrier = pltpu.get_barrier_semaphore()
    for nbr in (left, right):
        pl.semaphore_signal(barrier, inc=1, device_id=(nbr,),
                            device_id_type=pl.DeviceIdType.MESH)
    pl.semaphore_wait(barrier, 2)

    seed = pltpu.make_async_copy(
        partial_ref.at[pl.ds(my * MC, MC), :], buf_ref.at[0],
        local_sems.at[0])
    seed.start()
    seed.wait()

    def ring_send(g):
        rdma = pltpu.make_async_remote_copy(
            src_ref=buf_ref.at[g % 2],
            dst_ref=buf_ref.at[(g + 1) % 2],
            send_sem=send_sems.at[g % 2],
            recv_sem=recv_sems.at[(g + 1) % 2],
            device_id=(right,),
            device_id_type=pl.DeviceIdType.MESH,
        )
        if g >= 1:
            pl.semaphore_wait(credit_sem, 1)
        rdma.start()
        return rdma

    def ring_credit(g):
        if g < 2 * (N_DEV - 1) - 1:
            pl.semaphore_signal(credit_sem, inc=1, device_id=(left,),
                                device_id_type=pl.DeviceIdType.MESH)

    for s in range(N_DEV - 1):
        rdma = ring_send(s)
        c = lax.rem(my + (N_DEV - 1 - s), N_DEV)
        load = pltpu.make_async_copy(
            partial_ref.at[pl.ds(c * MC, MC), :], pchunk_ref,
            local_sems.at[0])
        load.start()
        rdma.wait_recv()
        load.wait()
        buf_ref[(s + 1) % 2] = buf_ref[(s + 1) % 2] + pchunk_ref[...]
        rdma.wait_send()
        ring_credit(s)

    own = lax.rem(my + 1, N_DEV)
    scale = sx_ref[0] * sw_ref[0]
    buf_ref[1] = jnp.maximum(buf_ref[1] * scale, 0.0)
    st = pltpu.make_async_copy(
        buf_ref.at[1], out_ref.at[pl.ds(own * MC, MC), :], local_sems.at[1])
    st.start()
    st.wait()

    for t in range(N_DEV - 1):
        g = (N_DEV - 1) + t
        rdma = ring_send(g)
        cr = lax.rem(my + (N_DEV - t), N_DEV)
        rdma.wait_recv()
        st = pltpu.make_async_copy(
            buf_ref.at[(g + 1) % 2], out_ref.at[pl.ds(cr * MC, MC), :],
            local_sems.at[1])
        st.start()
        st.wait()
        rdma.wait_send()
        ring_credit(g)


def _allreduce(partial, scale_x, scale_w):
    return pl.pallas_call(
        _ar_body,
        out_shape=jax.ShapeDtypeStruct((M, N), jnp.float32),
        in_specs=[
            pl.BlockSpec(memory_space=pl.ANY),
            pl.BlockSpec(memory_space=pltpu.SMEM),
            pl.BlockSpec(memory_space=pltpu.SMEM),
        ],
        out_specs=pl.BlockSpec(memory_space=pl.ANY),
        scratch_shapes=[
            pltpu.VMEM((2, MC, N), jnp.float32),
            pltpu.VMEM((MC, N), jnp.float32),
            pltpu.SemaphoreType.DMA((2,)),
            pltpu.SemaphoreType.DMA((2,)),
            pltpu.SemaphoreType.DMA((2,)),
            pltpu.SemaphoreType.REGULAR,
        ],
        compiler_params=pltpu.CompilerParams(
            collective_id=0, vmem_limit_bytes=100 * 1024 * 1024),
    )(partial, scale_x, scale_w)


def kernel(x, w_mat, scale_x, scale_w):
    partial = lax.dot_general(
        x.astype(jnp.bfloat16), w_mat.astype(jnp.bfloat16),
        (((1,), (0,)), ((), ())), preferred_element_type=jnp.float32)
    return _allreduce(partial, scale_x.astype(jnp.float32),
                      scale_w.astype(jnp.float32))


# device time: 790240 ns/iter; 3.5054x vs baseline; 3.5054x over previous
import jax
import jax.numpy as jnp
from jax import lax
from jax.experimental import pallas as pl
from jax.experimental.pallas import tpu as pltpu

N_DEV = 8
M, N = 4096, 8192
MC = M // N_DEV
NS = 4
QN = N // NS
RIGHTWARD = (True, True, False, False)
STEPS = 2 * (N_DEV - 1)


def _ar_body(partial_ref, sx_ref, sw_ref, out_ref,
             bufs, pch, send_sems, recv_sems, load_sems, store_sems,
             credit_sems):
    my = lax.axis_index("i")
    left = lax.rem(my + N_DEV - 1, N_DEV)
    right = lax.rem(my + 1, N_DEV)
    scale = sx_ref[0] * sw_ref[0]

    barrier = pltpu.get_barrier_semaphore()
    for nbr in (left, right):
        pl.semaphore_signal(barrier, inc=1, device_id=(nbr,),
                            device_id_type=pl.DeviceIdType.MESH)
    pl.semaphore_wait(barrier, 2)

    def dst_dev(k):
        return right if RIGHTWARD[k] else left

    def upstream(k):
        return left if RIGHTWARD[k] else right

    def rs_chunk(k, s):
        off = (N_DEV - 1 - s) if RIGHTWARD[k] else (s + 1)
        return lax.rem(my + off, N_DEV)

    def ag_chunk(k, t):
        off = (N_DEV - t) if RIGHTWARD[k] else t
        return lax.rem(my + off, N_DEV)

    def descr(k, g):
        return pltpu.make_async_remote_copy(
            src_ref=bufs.at[k, g % 2],
            dst_ref=bufs.at[k, (g + 1) % 2],
            send_sem=send_sems.at[k, g % 2],
            recv_sem=recv_sems.at[k, (g + 1) % 2],
            device_id=(dst_dev(k),),
            device_id_type=pl.DeviceIdType.MESH,
        )

    loads = {}
    stores = {}

    def start_load(k, c):
        cp = pltpu.make_async_copy(
            partial_ref.at[pl.ds(c * MC, MC), pl.ds(k * QN, QN)],
            pch.at[k], load_sems.at[k])
        cp.start()
        loads[k] = cp

    def wait_load(k):
        loads.pop(k).wait()

    def start_store(k, c):
        cp = pltpu.make_async_copy(
            pch.at[k], out_ref.at[pl.ds(c * MC, MC), pl.ds(k * QN, QN)],
            store_sems.at[k])
        cp.start()
        stores[k] = cp

    def wait_store(k):
        stores.pop(k).wait()

    for k in range(NS):
        start_load(k, my)
    for k in range(NS):
        wait_load(k)
        bufs[k, 0] = pch[k].astype(jnp.bfloat16)
    for k in range(NS):
        descr(k, 0).start()
        start_load(k, rs_chunk(k, 0))

    for g in range(STEPS):
        for k in range(NS):
            rd = descr(k, g)
            rd.wait_recv()
            slot = (g + 1) % 2
            if g < N_DEV - 2:
                wait_load(k)
                bufs[k, slot] = (bufs[k, slot].astype(jnp.float32)
                                 + pch[k]).astype(jnp.bfloat16)
            elif g == N_DEV - 2:
                wait_load(k)
                y = jnp.maximum(
                    (bufs[k, slot].astype(jnp.float32) + pch[k]) * scale, 0.0)
                bufs[k, slot] = y.astype(jnp.bfloat16)
                pch[k] = y
                start_store(k, rs_chunk(k, g))
            else:
                wait_store(k)
                pch[k] = bufs[k, slot].astype(jnp.float32)
                start_store(k, ag_chunk(k, g - (N_DEV - 1)))
            rd.wait_send()
            if g < STEPS - 1:
                pl.semaphore_signal(credit_sems.at[k], inc=1,
                                    device_id=(upstream(k),),
                                    device_id_type=pl.DeviceIdType.MESH)
                pl.semaphore_wait(credit_sems.at[k], 1)
                descr(k, g + 1).start()
                if g + 1 < N_DEV - 1:
                    start_load(k, rs_chunk(k, g + 1))

    for k in range(NS):
        wait_store(k)


def _allreduce(partial, scale_x, scale_w):
    return pl.pallas_call(
        _ar_body,
        out_shape=jax.ShapeDtypeStruct((M, N), jnp.float32),
        in_specs=[
            pl.BlockSpec(memory_space=pl.ANY),
            pl.BlockSpec(memory_space=pltpu.SMEM),
            pl.BlockSpec(memory_space=pltpu.SMEM),
        ],
        out_specs=pl.BlockSpec(memory_space=pl.ANY),
        scratch_shapes=[
            pltpu.VMEM((NS, 2, MC, QN), jnp.bfloat16),
            pltpu.VMEM((NS, MC, QN), jnp.float32),
            pltpu.SemaphoreType.DMA((NS, 2)),
            pltpu.SemaphoreType.DMA((NS, 2)),
            pltpu.SemaphoreType.DMA((NS,)),
            pltpu.SemaphoreType.DMA((NS,)),
            pltpu.SemaphoreType.REGULAR((NS,)),
        ],
        compiler_params=pltpu.CompilerParams(
            collective_id=0, vmem_limit_bytes=100 * 1024 * 1024),
    )(partial, scale_x, scale_w)


def kernel(x, w_mat, scale_x, scale_w):
    partial = lax.dot_general(
        x.astype(jnp.bfloat16), w_mat.astype(jnp.bfloat16),
        (((1,), (0,)), ((), ())), preferred_element_type=jnp.float32)
    return _allreduce(partial, scale_x.astype(jnp.float32),
                      scale_w.astype(jnp.float32))


# device time: 752044 ns/iter; 3.6835x vs baseline; 1.0508x over previous
import jax
import jax.numpy as jnp
from jax import lax
from jax.experimental import pallas as pl
from jax.experimental.pallas import tpu as pltpu

N_DEV = 8
M, N = 4096, 8192
MC = M // N_DEV
NS = 4
QN = N // NS
RIGHTWARD = (True, True, False, False)
STEPS = 2 * (N_DEV - 1)


def _ar_body(xb_ref, wb_ref, sx_ref, sw_ref, out_ref,
             bufs, pch, send_sems, recv_sems, store_sems, credit_sems):
    my = lax.axis_index("i")
    left = lax.rem(my + N_DEV - 1, N_DEV)
    right = lax.rem(my + 1, N_DEV)
    scale = sx_ref[0] * sw_ref[0]

    barrier = pltpu.get_barrier_semaphore()
    for nbr in (left, right):
        pl.semaphore_signal(barrier, inc=1, device_id=(nbr,),
                            device_id_type=pl.DeviceIdType.MESH)
    pl.semaphore_wait(barrier, 2)

    def dst_dev(k):
        return right if RIGHTWARD[k] else left

    def upstream(k):
        return left if RIGHTWARD[k] else right

    def rs_chunk(k, s):
        off = (N_DEV - 1 - s) if RIGHTWARD[k] else (s + 1)
        return lax.rem(my + off, N_DEV)

    def ag_chunk(k, t):
        off = (N_DEV - t) if RIGHTWARD[k] else t
        return lax.rem(my + off, N_DEV)

    def pdot(k, c):
        return jnp.dot(xb_ref[pl.ds(c * MC, MC), :],
                       wb_ref[:, k * QN:(k + 1) * QN],
                       preferred_element_type=jnp.float32)

    def descr(k, g):
        return pltpu.make_async_remote_copy(
            src_ref=bufs.at[k, g % 2],
            dst_ref=bufs.at[k, (g + 1) % 2],
            send_sem=send_sems.at[k, g % 2],
            recv_sem=recv_sems.at[k, (g + 1) % 2],
            device_id=(dst_dev(k),),
            device_id_type=pl.DeviceIdType.MESH,
        )

    stores = {}

    def start_store(k, c):
        cp = pltpu.make_async_copy(
            pch.at[k], out_ref.at[pl.ds(c * MC, MC), pl.ds(k * QN, QN)],
            store_sems.at[k])
        cp.start()
        stores[k] = cp

    def wait_store(k):
        stores.pop(k).wait()

    for k in range(NS):
        bufs[k, 0] = pdot(k, my).astype(jnp.bfloat16)
        descr(k, 0).start()

    for g in range(STEPS):
        for k in range(NS):
            rd = descr(k, g)
            rd.wait_recv()
            slot = (g + 1) % 2
            if g < N_DEV - 2:
                bufs[k, slot] = (bufs[k, slot].astype(jnp.float32)
                                 + pdot(k, rs_chunk(k, g))).astype(jnp.bfloat16)
            elif g == N_DEV - 2:
                y = jnp.maximum(
                    (bufs[k, slot].astype(jnp.float32)
                     + pdot(k, rs_chunk(k, g))) * scale, 0.0)
                bufs[k, slot] = y.astype(jnp.bfloat16)
                pch[k] = y
                start_store(k, rs_chunk(k, g))
            else:
                wait_store(k)
                pch[k] = bufs[k, slot].astype(jnp.float32)
                start_store(k, ag_chunk(k, g - (N_DEV - 1)))
            rd.wait_send()
            if g < STEPS - 1:
                pl.semaphore_signal(credit_sems.at[k], inc=1,
                                    device_id=(upstream(k),),
                                    device_id_type=pl.DeviceIdType.MESH)
                pl.semaphore_wait(credit_sems.at[k], 1)
                descr(k, g + 1).start()

    for k in range(NS):
        wait_store(k)


def _fused(xb, wb, scale_x, scale_w):
    return pl.pallas_call(
        _ar_body,
        out_shape=jax.ShapeDtypeStruct((M, N), jnp.float32),
        in_specs=[
            pl.BlockSpec(memory_space=pltpu.VMEM),
            pl.BlockSpec(memory_space=pltpu.VMEM),
            pl.BlockSpec(memory_space=pltpu.SMEM),
            pl.BlockSpec(memory_space=pltpu.SMEM),
        ],
        out_specs=pl.BlockSpec(memory_space=pl.ANY),
        scratch_shapes=[
            pltpu.VMEM((NS, 2, MC, QN), jnp.bfloat16),
            pltpu.VMEM((NS, MC, QN), jnp.float32),
            pltpu.SemaphoreType.DMA((NS, 2)),
            pltpu.SemaphoreType.DMA((NS, 2)),
            pltpu.SemaphoreType.DMA((NS,)),
            pltpu.SemaphoreType.REGULAR((NS,)),
        ],
        compiler_params=pltpu.CompilerParams(
            collective_id=0, vmem_limit_bytes=100 * 1024 * 1024),
    )(xb, wb, scale_x, scale_w)


def kernel(x, w_mat, scale_x, scale_w):
    return _fused(x.astype(jnp.bfloat16), w_mat.astype(jnp.bfloat16),
                  scale_x.astype(jnp.float32), scale_w.astype(jnp.float32))


# device time: 751959 ns/iter; 3.6839x vs baseline; 1.0001x over previous
import jax
import jax.numpy as jnp
from jax import lax
from jax.experimental import pallas as pl
from jax.experimental.pallas import tpu as pltpu

N_DEV = 8
M, N = 4096, 8192
MC = M // N_DEV
NS = 4
QN = N // NS
RIGHTWARD = (True, True, False, False)
STEPS = 2 * (N_DEV - 1)


def _ar_body(xb_ref, wb_ref, sx_ref, sw_ref, out_ref,
             bufs, pch, send_sems, recv_sems, store_sems, credit_sems):
    my = lax.axis_index("i")
    left = lax.rem(my + N_DEV - 1, N_DEV)
    right = lax.rem(my + 1, N_DEV)
    scale = sx_ref[0] * sw_ref[0]

    barrier = pltpu.get_barrier_semaphore()
    for nbr in (left, right):
        pl.semaphore_signal(barrier, inc=1, device_id=(nbr,),
                            device_id_type=pl.DeviceIdType.MESH)
    pl.semaphore_wait(barrier, 2)

    def dst_dev(k):
        return right if RIGHTWARD[k] else left

    def upstream(k):
        return left if RIGHTWARD[k] else right

    def rs_chunk(k, s):
        off = (N_DEV - 1 - s) if RIGHTWARD[k] else (s + 1)
        return lax.rem(my + off, N_DEV)

    def ag_chunk(k, t):
        off = (N_DEV - t) if RIGHTWARD[k] else t
        return lax.rem(my + off, N_DEV)

    def pdot(k, c):
        return jnp.dot(xb_ref[pl.ds(c * MC, MC), :],
                       wb_ref[:, k * QN:(k + 1) * QN],
                       preferred_element_type=jnp.float32)

    def descr(k, g):
        return pltpu.make_async_remote_copy(
            src_ref=bufs.at[k, g % 2],
            dst_ref=bufs.at[k, (g + 1) % 2],
            send_sem=send_sems.at[k, g % 2],
            recv_sem=recv_sems.at[k, (g + 1) % 2],
            device_id=(dst_dev(k),),
            device_id_type=pl.DeviceIdType.MESH,
        )

    stores = {}

    def start_store(k, c):
        cp = pltpu.make_async_copy(
            pch.at[k], out_ref.at[pl.ds(c * MC, MC), pl.ds(k * QN, QN)],
            store_sems.at[k])
        cp.start()
        stores[k] = cp

    def wait_store(k):
        stores.pop(k).wait()

    for k in range(NS):
        bufs[k, 0] = pdot(k, my).astype(jnp.bfloat16)
        descr(k, 0).start()

    for g in range(STEPS):
        for k in range(NS):
            rd = descr(k, g)
            rd.wait_recv()
            slot = (g + 1) % 2
            if g < N_DEV - 2:
                bufs[k, slot] = (bufs[k, slot]
                                 + pdot(k, rs_chunk(k, g)).astype(jnp.bfloat16))
            elif g == N_DEV - 2:
                y = jnp.maximum(
                    (bufs[k, slot].astype(jnp.float32)
                     + pdot(k, rs_chunk(k, g))) * scale, 0.0)
                bufs[k, slot] = y.astype(jnp.bfloat16)
                pch[k] = y
                start_store(k, rs_chunk(k, g))
            else:
                wait_store(k)
                pch[k] = bufs[k, slot].astype(jnp.float32)
                start_store(k, ag_chunk(k, g - (N_DEV - 1)))
            rd.wait_send()
            if g < STEPS - 1:
                pl.semaphore_signal(credit_sems.at[k], inc=1,
                                    device_id=(upstream(k),),
                                    device_id_type=pl.DeviceIdType.MESH)
                pl.semaphore_wait(credit_sems.at[k], 1)
                descr(k, g + 1).start()

    for k in range(NS):
        wait_store(k)


def _fused(xb, wb, scale_x, scale_w):
    return pl.pallas_call(
        _ar_body,
        out_shape=jax.ShapeDtypeStruct((M, N), jnp.float32),
        in_specs=[
            pl.BlockSpec(memory_space=pltpu.VMEM),
            pl.BlockSpec(memory_space=pltpu.VMEM),
            pl.BlockSpec(memory_space=pltpu.SMEM),
            pl.BlockSpec(memory_space=pltpu.SMEM),
        ],
        out_specs=pl.BlockSpec(memory_space=pl.ANY),
        scratch_shapes=[
            pltpu.VMEM((NS, 2, MC, QN), jnp.bfloat16),
            pltpu.VMEM((NS, MC, QN), jnp.float32),
            pltpu.SemaphoreType.DMA((NS, 2)),
            pltpu.SemaphoreType.DMA((NS, 2)),
            pltpu.SemaphoreType.DMA((NS,)),
            pltpu.SemaphoreType.REGULAR((NS,)),
        ],
        compiler_params=pltpu.CompilerParams(
            collective_id=0, vmem_limit_bytes=100 * 1024 * 1024),
    )(xb, wb, scale_x, scale_w)


def kernel(x, w_mat, scale_x, scale_w):
    return _fused(x.astype(jnp.bfloat16), w_mat.astype(jnp.bfloat16),
                  scale_x.astype(jnp.float32), scale_w.astype(jnp.float32))
